# Initial kernel scaffold; baseline (speedup 1.0000x reference)
import jax
import jax.numpy as jnp
from jax.experimental import pallas as pl


def kernel(x, adj, W_lin, W_layers, W_atts, W_c1, W_c2):
    raise NotImplementedError("write your pallas kernel here")


# speedup vs baseline: 2295.1085x; 2295.1085x over previous
import functools

import jax
import jax.numpy as jnp
from jax import lax
from jax.experimental import pallas as pl

N = 1024
IN = 256
OUT = 128
NH = 4
HD = OUT // NH


def _sem_kernel(x_ref, adj_ref, wlin_ref, wlay_ref, watt_ref, out_ref):
    x = x_ref[...]
    adj = adj_ref[...]

    h = lax.dot_general(x, wlin_ref[...], (((1,), (1,)), ((), ())),
                        preferred_element_type=jnp.float32)

    watt = watt_ref[...].reshape(NH, 2 * OUT)
    a = watt[:, :OUT]
    b = watt[:, OUT:]
    s = lax.dot_general(h, a, (((1,), (1,)), ((), ())),
                        preferred_element_type=jnp.float32)
    t_rows = lax.dot_general(b, h, (((1,), (1,)), ((), ())),
                             preferred_element_type=jnp.float32)

    wlay = wlay_ref[...].reshape(OUT, IN)
    xw = lax.dot_general(x, wlay, (((1,), (1,)), ((), ())),
                         preferred_element_type=jnp.float32)

    nz = (adj != 0.0).astype(jnp.float32)
    pad = jnp.float32(N * N) - jnp.sum(nz)
    rows0 = (lax.broadcasted_iota(jnp.int32, (N, 1), 0) == 0).astype(jnp.float32)

    for k in range(NH):
        sk = s[:, k:k + 1]
        tk_row = t_rows[k:k + 1, :]
        ak = jax.nn.sigmoid(sk + tk_row) * nz
        xwk = xw[:, k * HD:(k + 1) * HD]
        ok = lax.dot_general(ak, xwk, (((1,), (0,)), ((), ())),
                             preferred_element_type=jnp.float32)
        e00 = jax.nn.sigmoid(sk[0:1, :] + tk_row[:, 0:1])
        ok = ok + (pad * e00) * rows0 * xwk[0:1, :]
        out_ref[:, k * HD:(k + 1) * HD] = ok


@jax.jit
def kernel(x, adj, W_lin, W_layers, W_atts, W_c1, W_c2):
    del W_c1, W_c2
    return pl.pallas_call(
        _sem_kernel,
        out_shape=jax.ShapeDtypeStruct((N, OUT), jnp.float32),
    )(x, adj, W_lin, W_layers, W_atts)
